# baseline (speedup 1.0000x reference)
import jax
import jax.numpy as jnp
from jax import lax
from jax.experimental import pallas as pl
from jax.experimental.pallas import tpu as pltpu, tpu_sc as plsc

N_NODES = 10000
N_EDGES = 160000
D_FEAT = 256

NC = 2
NS = 16
DH = D_FEAT // NC
EPT = N_EDGES // NS
CHUNK = 40
CPAD = 48
NCHUNK = EPT // CHUNK
RPT = N_NODES // NS
RBLK = 25
NRB = RPT // RBLK


def _gcn_body(emb_hbm, col_hbm, row_hbm, vals_hbm, out_hbm,
              col_v, row_v, vals_v, rows0, rows1, ob0, ob1, acc,
              sem0, sem1, osem0, osem1):
    c = lax.axis_index("c")
    s = lax.axis_index("s")

    ebase = s * EPT
    pltpu.async_copy(col_hbm.at[pl.ds(ebase, EPT)], col_v, sem1)
    pltpu.async_copy(row_hbm.at[pl.ds(ebase, EPT)], row_v, sem1)
    pltpu.async_copy(vals_hbm.at[pl.ds(ebase, EPT)],
                     vals_v.at[pl.ds(0, EPT)], sem1)

    def _zset(i, _):
        for g in range(DH // 16):
            ob0[i, pl.ds(g * 16, 16)] = jnp.zeros((16,), jnp.float32)
        return 0
    lax.fori_loop(0, RBLK, _zset, 0)
    for b in range(NRB):
        pltpu.async_copy(ob0, acc.at[pl.ds(s * RPT + b * RBLK, RBLK), :],
                         sem0)
    for b in range(NRB):
        pltpu.make_async_copy(ob0, acc.at[pl.ds(s * RPT, RBLK), :],
                              sem0).wait()
    pltpu.make_async_copy(col_hbm.at[pl.ds(0, EPT)], col_v, sem1).wait()
    pltpu.make_async_copy(row_hbm.at[pl.ds(0, EPT)], row_v, sem1).wait()
    pltpu.make_async_copy(vals_hbm.at[pl.ds(0, EPT)],
                          vals_v.at[pl.ds(0, EPT)], sem1).wait()
    plsc.subcore_barrier()

    emb_c = emb_hbm.at[c]

    def _gather(ch, buf, sem):
        idx = col_v.at[pl.ds(ch * CHUNK, CHUNK)]
        pltpu.async_copy(emb_c.at[idx], buf.at[pl.ds(0, CHUNK), :], sem)

    def _wait(buf, sem):
        pltpu.make_async_copy(emb_c.at[col_v.at[pl.ds(0, CHUNK)]],
                              buf.at[pl.ds(0, CHUNK), :], sem).wait()

    def _compute_scatter(ch, buf):
        for jo in range(0, CPAD, 16):
            vblk = vals_v[pl.ds(ch * CHUNK + jo, 16)]
            for ji in range(16):
                j = jo + ji
                vv = vblk[ji]
                for g in range(DH // 16):
                    sl = pl.ds(g * 16, 16)
                    buf[j, sl] = buf[j, sl] * vv
        pltpu.sync_copy(buf.at[pl.ds(0, CHUNK), :],
                        acc.at[row_v.at[pl.ds(ch * CHUNK, CHUNK)]],
                        add=True)

    _gather(0, rows0, sem0)
    _gather(1, rows1, sem1)

    def _pair(i, _):
        ch0 = 2 * i
        _wait(rows0, sem0)
        _compute_scatter(ch0, rows0)

        @pl.when(i < NCHUNK // 2 - 1)
        def _():
            _gather(ch0 + 2, rows0, sem0)
        _wait(rows1, sem1)
        _compute_scatter(ch0 + 1, rows1)

        @pl.when(i < NCHUNK // 2 - 1)
        def _():
            _gather(ch0 + 3, rows1, sem1)
        return 0
    lax.fori_loop(0, NCHUNK // 2, _pair, 0)
    plsc.subcore_barrier()

    rbase = s * RPT

    def _ain(b, ob, isem):
        pltpu.async_copy(acc.at[pl.ds(rbase + b * RBLK, RBLK), :], ob, isem)

    def _iwait(ob, isem):
        pltpu.make_async_copy(acc.at[pl.ds(rbase, RBLK), :], ob, isem).wait()

    def _aout(b, ob, osem):
        pltpu.async_copy(
            ob,
            out_hbm.at[pl.ds(rbase + b * RBLK, RBLK), pl.ds(c * DH, DH)],
            osem)

    def _owait(ob, osem):
        pltpu.make_async_copy(
            ob, out_hbm.at[pl.ds(rbase, RBLK), pl.ds(c * DH, DH)],
            osem).wait()

    def _lrelu(ob):
        def body(i, _):
            for g in range(DH // 16):
                sl = pl.ds(g * 16, 16)
                x = ob[i, sl]
                ob[i, sl] = jnp.maximum(x, x * 0.5)
            return 0
        lax.fori_loop(0, RBLK, body, 0)

    _ain(0, ob0, sem0)
    _ain(1, ob1, sem1)

    def _dpair(i, _):
        b0 = 2 * i
        _iwait(ob0, sem0)
        _lrelu(ob0)
        _aout(b0, ob0, osem0)
        _iwait(ob1, sem1)
        _lrelu(ob1)
        _aout(b0 + 1, ob1, osem1)
        _owait(ob0, osem0)
        _ain(b0 + 2, ob0, sem0)

        @pl.when(i < NRB // 2 - 1)
        def _():
            _owait(ob1, osem1)
            _ain(b0 + 3, ob1, sem1)
        return 0
    lax.fori_loop(0, NRB // 2, _dpair, 0)

    _iwait(ob0, sem0)
    _lrelu(ob0)
    _aout(NRB - 1, ob0, osem0)
    _owait(ob0, osem0)
    _owait(ob1, osem1)


def kernel(edge_index, edge_vals, embeds):
    emb_split = embeds.reshape(N_NODES, NC, DH).transpose(1, 0, 2)
    col = edge_index[1]
    row = edge_index[0]

    k = pl.kernel(
        _gcn_body,
        out_type=jax.ShapeDtypeStruct((N_NODES, D_FEAT), jnp.float32),
        mesh=plsc.VectorSubcoreMesh(core_axis_name="c", subcore_axis_name="s"),
        compiler_params=pltpu.CompilerParams(use_tc_tiling_on_sc=False),
        scratch_types=[
            pltpu.VMEM((EPT,), jnp.int32),
            pltpu.VMEM((EPT,), jnp.int32),
            pltpu.VMEM((EPT + 16,), jnp.float32),
            pltpu.VMEM((CPAD, DH), jnp.float32),
            pltpu.VMEM((CPAD, DH), jnp.float32),
            pltpu.VMEM((RBLK, DH), jnp.float32),
            pltpu.VMEM((RBLK, DH), jnp.float32),
            pltpu.VMEM_SHARED((N_NODES, DH), jnp.float32),
            pltpu.SemaphoreType.DMA,
            pltpu.SemaphoreType.DMA,
            pltpu.SemaphoreType.DMA,
            pltpu.SemaphoreType.DMA,
        ],
    )
    return k(emb_split, col, row, edge_vals)


# speedup vs baseline: 1.2009x; 1.2009x over previous
import jax
import jax.numpy as jnp
from jax import lax
from jax.experimental import pallas as pl
from jax.experimental.pallas import tpu as pltpu, tpu_sc as plsc

N_NODES = 10000
N_EDGES = 160000
D_FEAT = 256

NC = 2
NS = 16
DH = D_FEAT // NC
EPT = N_EDGES // NS
CHUNK = 80
NCHUNK = EPT // CHUNK
RPT = N_NODES // NS
RBLK = 25
NRB = RPT // RBLK


def _gcn_body(emb_hbm, col_hbm, row_hbm, vals_hbm, out_hbm,
              col_v, row_v, vals_v, rows0, rows1, acc,
              sem0, sem1, osem0, osem1):
    c = lax.axis_index("c")
    s = lax.axis_index("s")
    ob0 = rows0.at[pl.ds(0, RBLK), :]
    ob1 = rows1.at[pl.ds(0, RBLK), :]

    ebase = s * EPT
    pltpu.async_copy(col_hbm.at[pl.ds(ebase, EPT)], col_v, sem1)
    pltpu.async_copy(row_hbm.at[pl.ds(ebase, EPT)], row_v, sem1)
    pltpu.async_copy(vals_hbm.at[pl.ds(ebase, EPT)], vals_v, sem1)

    def _zset(i, _):
        for g in range(DH // 16):
            rows0[i, pl.ds(g * 16, 16)] = jnp.zeros((16,), jnp.float32)
        return 0
    lax.fori_loop(0, RBLK, _zset, 0)
    for b in range(NRB):
        pltpu.async_copy(ob0, acc.at[pl.ds(s * RPT + b * RBLK, RBLK), :],
                         sem0)
    for b in range(NRB):
        pltpu.make_async_copy(ob0, acc.at[pl.ds(s * RPT, RBLK), :],
                              sem0).wait()
    pltpu.make_async_copy(col_hbm.at[pl.ds(0, EPT)], col_v, sem1).wait()
    pltpu.make_async_copy(row_hbm.at[pl.ds(0, EPT)], row_v, sem1).wait()
    pltpu.make_async_copy(vals_hbm.at[pl.ds(0, EPT)], vals_v, sem1).wait()
    plsc.subcore_barrier()

    emb_c = emb_hbm.at[c]

    def _gather(ch, buf, sem):
        idx = col_v.at[pl.ds(ch * CHUNK, CHUNK)]
        pltpu.async_copy(emb_c.at[idx], buf, sem)

    def _wait(buf, sem):
        pltpu.make_async_copy(emb_c.at[col_v.at[pl.ds(0, CHUNK)]], buf,
                              sem).wait()

    def _compute_scatter(ch, buf):
        for jo in range(0, CHUNK, 16):
            vblk = vals_v[pl.ds(ch * CHUNK + jo, 16)]
            for ji in range(16):
                j = jo + ji
                vv = vblk[ji]
                for g in range(DH // 16):
                    sl = pl.ds(g * 16, 16)
                    buf[j, sl] = buf[j, sl] * vv
        pltpu.sync_copy(buf, acc.at[row_v.at[pl.ds(ch * CHUNK, CHUNK)]],
                        add=True)

    _gather(0, rows0, sem0)
    _gather(1, rows1, sem1)

    def _pair(i, _):
        ch0 = 2 * i
        _wait(rows0, sem0)
        _compute_scatter(ch0, rows0)
        _gather(ch0 + 2, rows0, sem0)
        _wait(rows1, sem1)
        _compute_scatter(ch0 + 1, rows1)

        @pl.when(i < (NCHUNK - 1) // 2 - 1)
        def _():
            _gather(ch0 + 3, rows1, sem1)
        return 0
    lax.fori_loop(0, (NCHUNK - 1) // 2, _pair, 0)

    _wait(rows0, sem0)
    _compute_scatter(NCHUNK - 1, rows0)
    plsc.subcore_barrier()

    rbase = s * RPT

    def _ain(b, ob, isem):
        pltpu.async_copy(acc.at[pl.ds(rbase + b * RBLK, RBLK), :], ob, isem)

    def _iwait(ob, isem):
        pltpu.make_async_copy(acc.at[pl.ds(rbase, RBLK), :], ob, isem).wait()

    def _aout(b, ob, osem):
        pltpu.async_copy(
            ob,
            out_hbm.at[pl.ds(rbase + b * RBLK, RBLK), pl.ds(c * DH, DH)],
            osem)

    def _owait(ob, osem):
        pltpu.make_async_copy(
            ob, out_hbm.at[pl.ds(rbase, RBLK), pl.ds(c * DH, DH)],
            osem).wait()

    def _lrelu(base):
        def body(i, _):
            for g in range(DH // 16):
                sl = pl.ds(g * 16, 16)
                x = base[i, sl]
                base[i, sl] = jnp.maximum(x, x * 0.5)
            return 0
        lax.fori_loop(0, RBLK, body, 0)

    _ain(0, ob0, sem0)
    _ain(1, ob1, sem1)

    def _dpair(i, _):
        b0 = 2 * i
        _iwait(ob0, sem0)
        _lrelu(rows0)
        _aout(b0, ob0, osem0)
        _iwait(ob1, sem1)
        _lrelu(rows1)
        _aout(b0 + 1, ob1, osem1)
        _owait(ob0, osem0)
        _ain(b0 + 2, ob0, sem0)

        @pl.when(i < NRB // 2 - 1)
        def _():
            _owait(ob1, osem1)
            _ain(b0 + 3, ob1, sem1)
        return 0
    lax.fori_loop(0, NRB // 2, _dpair, 0)

    _iwait(ob0, sem0)
    _lrelu(rows0)
    _aout(NRB - 1, ob0, osem0)
    _owait(ob0, osem0)
    _owait(ob1, osem1)


def kernel(edge_index, edge_vals, embeds):
    emb_split = embeds.reshape(N_NODES, NC, DH).transpose(1, 0, 2)
    col = edge_index[1]
    row = edge_index[0]

    k = pl.kernel(
        _gcn_body,
        out_type=jax.ShapeDtypeStruct((N_NODES, D_FEAT), jnp.float32),
        mesh=plsc.VectorSubcoreMesh(core_axis_name="c", subcore_axis_name="s"),
        compiler_params=pltpu.CompilerParams(use_tc_tiling_on_sc=False),
        scratch_types=[
            pltpu.VMEM((EPT,), jnp.int32),
            pltpu.VMEM((EPT,), jnp.int32),
            pltpu.VMEM((EPT,), jnp.float32),
            pltpu.VMEM((CHUNK, DH), jnp.float32),
            pltpu.VMEM((CHUNK, DH), jnp.float32),
            pltpu.VMEM_SHARED((N_NODES, DH), jnp.float32),
            pltpu.SemaphoreType.DMA,
            pltpu.SemaphoreType.DMA,
            pltpu.SemaphoreType.DMA,
            pltpu.SemaphoreType.DMA,
        ],
    )
    return k(emb_split, col, row, edge_vals)


# speedup vs baseline: 1.2262x; 1.0210x over previous
import jax
import jax.numpy as jnp
from jax import lax
from jax.experimental import pallas as pl
from jax.experimental.pallas import tpu as pltpu, tpu_sc as plsc

N_NODES = 10000
N_EDGES = 160000
D_FEAT = 256

NC = 2
NS = 16
DH = D_FEAT // NC
EPT = N_EDGES // NS
CHUNK = 80
NCHUNK = EPT // CHUNK
RPT = N_NODES // NS
RBLK = 25
NRB = RPT // RBLK


def _gcn_body(emb_hbm, col_hbm, row_hbm, vals_hbm, out_hbm,
              col_v, row_v, valsb0, valsb1, rows0, rows1, ob0, ob1, acc,
              sem0, sem1, osem0, osem1):
    c = lax.axis_index("c")
    s = lax.axis_index("s")

    ebase = s * EPT
    pltpu.async_copy(col_hbm.at[pl.ds(ebase, EPT)], col_v, sem1)
    pltpu.async_copy(row_hbm.at[pl.ds(ebase, EPT)], row_v, sem1)

    def _zset(i, _):
        for g in range(DH // 16):
            ob0[i, pl.ds(g * 16, 16)] = jnp.zeros((16,), jnp.float32)
        return 0
    lax.fori_loop(0, RBLK, _zset, 0)
    for b in range(NRB):
        pltpu.async_copy(ob0, acc.at[pl.ds(s * RPT + b * RBLK, RBLK), :],
                         sem0)
    for b in range(NRB):
        pltpu.make_async_copy(ob0, acc.at[pl.ds(s * RPT, RBLK), :],
                              sem0).wait()
    pltpu.make_async_copy(col_hbm.at[pl.ds(0, EPT)], col_v, sem1).wait()
    pltpu.make_async_copy(row_hbm.at[pl.ds(0, EPT)], row_v, sem1).wait()
    plsc.subcore_barrier()

    emb_c = emb_hbm.at[c]

    def _gather(ch, buf, valsb, sem):
        idx = col_v.at[pl.ds(ch * CHUNK, CHUNK)]
        pltpu.async_copy(emb_c.at[idx], buf, sem)
        pltpu.async_copy(vals_hbm.at[pl.ds(ebase + ch * CHUNK, CHUNK)],
                         valsb, sem)

    def _wait(buf, valsb, sem):
        pltpu.make_async_copy(emb_c.at[col_v.at[pl.ds(0, CHUNK)]], buf,
                              sem).wait()
        pltpu.make_async_copy(vals_hbm.at[pl.ds(0, CHUNK)], valsb,
                              sem).wait()

    def _compute_scatter(ch, buf, valsb):
        for jo in range(0, CHUNK, 16):
            vblk = valsb[pl.ds(jo, 16)]
            for ji in range(16):
                j = jo + ji
                vv = vblk[ji]
                for g in range(DH // 16):
                    sl = pl.ds(g * 16, 16)
                    buf[j, sl] = buf[j, sl] * vv
        pltpu.sync_copy(buf, acc.at[row_v.at[pl.ds(ch * CHUNK, CHUNK)]],
                        add=True)

    _gather(0, rows0, valsb0, sem0)
    _gather(1, rows1, valsb1, sem1)

    def _pair(i, _):
        ch0 = 2 * i
        _wait(rows0, valsb0, sem0)
        _compute_scatter(ch0, rows0, valsb0)
        _gather(ch0 + 2, rows0, valsb0, sem0)
        _wait(rows1, valsb1, sem1)
        _compute_scatter(ch0 + 1, rows1, valsb1)

        @pl.when(i < (NCHUNK - 1) // 2 - 1)
        def _():
            _gather(ch0 + 3, rows1, valsb1, sem1)
        return 0
    lax.fori_loop(0, (NCHUNK - 1) // 2, _pair, 0)

    _wait(rows0, valsb0, sem0)
    _compute_scatter(NCHUNK - 1, rows0, valsb0)
    plsc.subcore_barrier()

    rbase = s * RPT

    def _ain(b, ob, isem):
        pltpu.async_copy(acc.at[pl.ds(rbase + b * RBLK, RBLK), :], ob, isem)

    def _iwait(ob, isem):
        pltpu.make_async_copy(acc.at[pl.ds(rbase, RBLK), :], ob, isem).wait()

    def _aout(b, ob, osem):
        pltpu.async_copy(
            ob,
            out_hbm.at[pl.ds(rbase + b * RBLK, RBLK), pl.ds(c * DH, DH)],
            osem)

    def _owait(ob, osem):
        pltpu.make_async_copy(
            ob, out_hbm.at[pl.ds(rbase, RBLK), pl.ds(c * DH, DH)],
            osem).wait()

    def _lrelu(ob):
        def body(i, _):
            for g in range(DH // 16):
                sl = pl.ds(g * 16, 16)
                x = ob[i, sl]
                ob[i, sl] = jnp.maximum(x, x * 0.5)
            return 0
        lax.fori_loop(0, RBLK, body, 0)

    _ain(0, ob0, sem0)
    _ain(1, ob1, sem1)

    def _dpair(i, _):
        b0 = 2 * i
        _iwait(ob0, sem0)
        _lrelu(ob0)
        _aout(b0, ob0, osem0)
        _iwait(ob1, sem1)
        _lrelu(ob1)
        _aout(b0 + 1, ob1, osem1)
        _owait(ob0, osem0)
        _ain(b0 + 2, ob0, sem0)

        @pl.when(i < NRB // 2 - 1)
        def _():
            _owait(ob1, osem1)
            _ain(b0 + 3, ob1, sem1)
        return 0
    lax.fori_loop(0, NRB // 2, _dpair, 0)

    _iwait(ob0, sem0)
    _lrelu(ob0)
    _aout(NRB - 1, ob0, osem0)
    _owait(ob0, osem0)
    _owait(ob1, osem1)


def kernel(edge_index, edge_vals, embeds):
    emb_split = embeds.reshape(N_NODES, NC, DH).transpose(1, 0, 2)
    col = edge_index[1]
    row = edge_index[0]

    k = pl.kernel(
        _gcn_body,
        out_type=jax.ShapeDtypeStruct((N_NODES, D_FEAT), jnp.float32),
        mesh=plsc.VectorSubcoreMesh(core_axis_name="c", subcore_axis_name="s"),
        compiler_params=pltpu.CompilerParams(use_tc_tiling_on_sc=False),
        scratch_types=[
            pltpu.VMEM((EPT,), jnp.int32),
            pltpu.VMEM((EPT,), jnp.int32),
            pltpu.VMEM((CHUNK,), jnp.float32),
            pltpu.VMEM((CHUNK,), jnp.float32),
            pltpu.VMEM((CHUNK, DH), jnp.float32),
            pltpu.VMEM((CHUNK, DH), jnp.float32),
            pltpu.VMEM((RBLK, DH), jnp.float32),
            pltpu.VMEM((RBLK, DH), jnp.float32),
            pltpu.VMEM_SHARED((N_NODES, DH), jnp.float32),
            pltpu.SemaphoreType.DMA,
            pltpu.SemaphoreType.DMA,
            pltpu.SemaphoreType.DMA,
            pltpu.SemaphoreType.DMA,
        ],
    )
    return k(emb_split, col, row, edge_vals)
